# Initial kernel scaffold; baseline (speedup 1.0000x reference)
import jax
import jax.numpy as jnp
from jax.experimental import pallas as pl


def kernel(x, edge_index, num_nodes, W1l, W1r, b1, W2l, W2r, b2):
    raise NotImplementedError("write your pallas kernel here")


# speedup vs baseline: 10.1128x; 10.1128x over previous
import functools

import jax
import jax.numpy as jnp
from jax import lax
from jax.experimental import pallas as pl
from jax.experimental.pallas import tpu as pltpu
from jax.experimental.pallas import tpu_sc as plsc

NC = 2
NS = 16
NW = NC * NS
CH = 80


def _sc_edge_agg(table, srcr, dstr, with_deg):
    n, d = table.shape
    nw, nch, ch = srcr.shape
    npad = ((n + 8 * NS - 1) // (8 * NS)) * (8 * NS)
    span = npad // NS

    zrow = jnp.zeros((span, d), jnp.float32)
    zdeg = jnp.zeros((span, 8), jnp.float32)
    ones = jnp.ones((ch, 8), jnp.float32)

    out_type = [jax.ShapeDtypeStruct((NC, npad, d), jnp.float32)]
    scratch = [
        pltpu.VMEM((nch, ch), jnp.int32),
        pltpu.VMEM((nch, ch), jnp.int32),
        pltpu.VMEM((ch, d), jnp.float32),
        pltpu.VMEM((ch, 8), jnp.float32),
        pltpu.VMEM_SHARED((npad, d), jnp.float32),
        pltpu.VMEM_SHARED((npad, 8), jnp.float32),
        pltpu.SemaphoreType.DMA,
    ]
    if with_deg:
        out_type.append(jax.ShapeDtypeStruct((NC, npad, 8), jnp.float32))

    mesh = plsc.VectorSubcoreMesh(core_axis_name="c", subcore_axis_name="s")

    @functools.partial(
        pl.kernel, mesh=mesh, out_type=out_type, scratch_types=scratch,
        compiler_params=pltpu.CompilerParams(use_tc_tiling_on_sc=False))
    def k(tbl, src_h, dst_h, z_h, zd_h, on_h, *rest):
        if with_deg:
            acc_out, deg_out = rest[0], rest[1]
            rest = rest[2:]
        else:
            acc_out = rest[0]
            rest = rest[1:]
        src_v, dst_v, rows_v, ones_v, acc_sh, deg_sh, sem = rest
        cid = lax.axis_index("c")
        sid = lax.axis_index("s")
        wid = cid * NS + sid
        base = sid * span
        pltpu.sync_copy(z_h, acc_sh.at[pl.ds(base, span)])
        if with_deg:
            pltpu.sync_copy(zd_h, deg_sh.at[pl.ds(base, span)])
            pltpu.sync_copy(on_h, ones_v)
        pltpu.sync_copy(src_h.at[wid], src_v)
        pltpu.sync_copy(dst_h.at[wid], dst_v)
        plsc.subcore_barrier()

        def step(j, carry):
            pltpu.async_copy(tbl.at[src_v.at[j]], rows_v, sem).wait()
            pltpu.sync_copy(rows_v, acc_sh.at[dst_v.at[j]], add=True)
            if with_deg:
                pltpu.sync_copy(ones_v, deg_sh.at[dst_v.at[j]], add=True)
            return carry

        lax.fori_loop(0, nch, step, 0)
        plsc.subcore_barrier()
        pltpu.sync_copy(acc_sh.at[pl.ds(base, span)],
                        acc_out.at[cid].at[pl.ds(base, span)])
        if with_deg:
            pltpu.sync_copy(deg_sh.at[pl.ds(base, span)],
                            deg_out.at[cid].at[pl.ds(base, span)])

    res = k(table, srcr, dstr, zrow, zdeg, ones)
    if with_deg:
        return res[0], res[1]
    return res[0] if isinstance(res, (list, tuple)) else res


def _tc0_body(x_ref, wl_ref, wr_ref, xl_ref, xr_ref):
    xb = x_ref[...]
    xl_ref[...] = jnp.dot(xb, wl_ref[...], preferred_element_type=jnp.float32)
    xr_ref[...] = jnp.dot(xb, wr_ref[...], preferred_element_type=jnp.float32)


def _tc1_body(xl_ref, acc_ref, deg_ref, b1_ref, w2r_ref, h_ref, hr8_ref):
    nrows = xl_ref.shape[0]
    acc = acc_ref[...]
    deg = deg_ref[...]
    s = (acc[0] + acc[1])[:nrows]
    dg = (deg[0] + deg[1])[:nrows, :1]
    inv = 1.0 / jnp.maximum(dg, 1.0)
    h = jax.nn.relu(xl_ref[...] + s * inv + b1_ref[...])
    h_ref[...] = h
    hr = jnp.dot(h, w2r_ref[...], preferred_element_type=jnp.float32)
    hr8_ref[...] = jnp.broadcast_to(hr, (h.shape[0], 8))


def _tc2_body(h_ref, w2l_ref, b2_ref, acc2_ref, deg_ref, out_ref):
    nrows = h_ref.shape[0]
    acc2 = acc2_ref[...]
    deg = deg_ref[...]
    a2 = (acc2[0] + acc2[1])[:nrows, :1]
    dg = (deg[0] + deg[1])[:nrows, :1]
    inv = 1.0 / jnp.maximum(dg, 1.0)
    hl = jnp.dot(h_ref[...], w2l_ref[...], preferred_element_type=jnp.float32)
    x2 = hl + a2 * inv + b2_ref[...]
    out_ref[...] = jnp.max(x2).reshape(1, 1)


def kernel(x, edge_index, num_nodes, W1l, W1r, b1, W2l, W2r, b2):
    n, in_dim = x.shape
    hid = W1l.shape[1]
    e = edge_index.shape[1]
    per_w = e // NW
    nch = per_w // CH

    src = edge_index[0].astype(jnp.int32).reshape(NW, nch, CH)
    dst = edge_index[1].astype(jnp.int32).reshape(NW, nch, CH)

    rb = 1000
    xl, xr = pl.pallas_call(
        _tc0_body,
        grid=(n // rb,),
        in_specs=[
            pl.BlockSpec((rb, in_dim), lambda i: (i, 0)),
            pl.BlockSpec((in_dim, hid), lambda i: (0, 0)),
            pl.BlockSpec((in_dim, hid), lambda i: (0, 0)),
        ],
        out_specs=[
            pl.BlockSpec((rb, hid), lambda i: (i, 0)),
            pl.BlockSpec((rb, hid), lambda i: (i, 0)),
        ],
        out_shape=[
            jax.ShapeDtypeStruct((n, hid), jnp.float32),
            jax.ShapeDtypeStruct((n, hid), jnp.float32),
        ],
    )(x, W1l, W1r)

    acc1, deg = _sc_edge_agg(xr, src, dst, with_deg=True)

    h, hr8 = pl.pallas_call(
        _tc1_body,
        out_shape=[
            jax.ShapeDtypeStruct((n, hid), jnp.float32),
            jax.ShapeDtypeStruct((n, 8), jnp.float32),
        ],
    )(xl, acc1, deg, b1.reshape(1, hid), W2r)

    acc2 = _sc_edge_agg(hr8, src, dst, with_deg=False)

    out = pl.pallas_call(
        _tc2_body,
        out_shape=jax.ShapeDtypeStruct((1, 1), jnp.float32),
    )(h, W2l, b2.reshape(1, 1), acc2, deg)

    return (out, h, h)
